# baseline (device time: 11748 ns/iter reference)
import jax
import jax.numpy as jnp
from jax import lax
from jax.experimental import pallas as pl
from jax.experimental.pallas import tpu as pltpu

N_DEV = 4
B, SQ, HQ, DH = 2, 128, 4, 64
QBLK = 64
HD = HQ * DH
HS = HQ * SQ


def kernel(x, Wq, K_ext, V_ext, Wo):
    kvT = jnp.concatenate(
        [K_ext.transpose(0, 2, 3, 1), V_ext.transpose(0, 2, 3, 1)], axis=0
    ).astype(jnp.bfloat16)

    def body(x_hbm, wq_hbm, kv_ref, wo_hbm, out_hbm,
             rk_ref, rv_ref, lk_ref, lv_ref, vstage_ref, vrem_ref, ctx_ref,
             wo_ref, x_ref, wq_ref, out_ref,
             send_sems, recv_sems, wo_sem, x_sem, wq_sem, out_sem):
        my = lax.axis_index("i")
        partner = (my + 2) % N_DEV

        x_copy = pltpu.make_async_copy(x_hbm, x_ref, x_sem)
        wq_copy = pltpu.make_async_copy(wq_hbm, wq_ref, wq_sem)
        wo_copy = pltpu.make_async_copy(wo_hbm, wo_ref, wo_sem)
        x_copy.start()
        wq_copy.start()
        wo_copy.start()

        rk_ref[...] = jnp.zeros((B, HD, HS), jnp.bfloat16)
        rv_ref[...] = jnp.zeros((B, HS, HD), jnp.bfloat16)

        barrier_sem = pltpu.get_barrier_semaphore()
        pl.semaphore_signal(
            barrier_sem, inc=1,
            device_id=(partner,), device_id_type=pl.DeviceIdType.MESH,
        )
        pl.semaphore_wait(barrier_sem, 1)

        k_rdmas = []
        for b in range(B):
            for h in range(HQ):
                i = b * HQ + h
                r = pltpu.make_async_remote_copy(
                    src_ref=kv_ref.at[b, h],
                    dst_ref=rk_ref.at[b, pl.ds(h * DH, DH),
                                      pl.ds(h * SQ, SQ)],
                    send_sem=send_sems.at[i], recv_sem=recv_sems.at[i],
                    device_id=(partner,), device_id_type=pl.DeviceIdType.MESH,
                )
                r.start()
                k_rdmas.append(r)

        for b in range(B):
            for h in range(HQ):
                vstage_ref[b, :, h * DH:(h + 1) * DH] = jnp.transpose(
                    kv_ref[B + b, h]
                )
        v_rdma = pltpu.make_async_remote_copy(
            src_ref=vstage_ref, dst_ref=vrem_ref,
            send_sem=send_sems.at[B * HQ], recv_sem=recv_sems.at[B * HQ],
            device_id=(partner,), device_id_type=pl.DeviceIdType.MESH,
        )
        v_rdma.start()

        lk_ref[...] = jnp.zeros((B, HD, HS), jnp.bfloat16)
        lv_ref[...] = jnp.zeros((B, HS, HD), jnp.bfloat16)
        for b in range(B):
            for h in range(HQ):
                lk_ref[b, h * DH:(h + 1) * DH, h * SQ:(h + 1) * SQ] = (
                    kv_ref[b, h]
                )
                lv_ref[b, h * SQ:(h + 1) * SQ, h * DH:(h + 1) * DH] = (
                    vstage_ref[b, :, h * DH:(h + 1) * DH]
                )

        qid = lax.broadcasted_iota(jnp.int32, (SQ, HS), 0) // QBLK
        cid = (lax.broadcasted_iota(jnp.int32, (SQ, HS), 1) // QBLK) % 2
        mask = (qid == cid).astype(jnp.float32)

        x_copy.wait()
        wq_copy.wait()
        q_all = jnp.dot(
            x_ref[...].reshape(B * SQ, 512).astype(jnp.bfloat16),
            wq_ref[...].astype(jnp.bfloat16),
            preferred_element_type=jnp.float32,
        )
        q_s = (q_all * 0.125).astype(jnp.bfloat16)

        def half(qb, kbd, vbd):
            s = jnp.dot(qb, kbd, preferred_element_type=jnp.float32)
            e = jnp.exp(s) * mask
            ds = [
                jnp.sum(e[:, h * SQ:(h + 1) * SQ], axis=-1, keepdims=True)
                for h in range(HQ)
            ]
            c = jnp.dot(
                e.astype(jnp.bfloat16), vbd,
                preferred_element_type=jnp.float32,
            )
            return ds, c

        loc = [
            half(q_s[b * SQ:(b + 1) * SQ, :], lk_ref[b], lv_ref[b])
            for b in range(B)
        ]

        for r in k_rdmas:
            r.wait_recv()
        v_rdma.wait_recv()
        for b in range(B):
            for h in range(HQ):
                rv_ref[b, h * SQ:(h + 1) * SQ, h * DH:(h + 1) * DH] = (
                    vrem_ref[b, :, h * DH:(h + 1) * DH]
                )
        for b in range(B):
            d_l, c_l = loc[b]
            d_r, c_r = half(q_s[b * SQ:(b + 1) * SQ, :], rk_ref[b], rv_ref[b])
            csum = c_l + c_r
            for h in range(HQ):
                inv = 1.0 / (d_l[h] + d_r[h])
                ctx_ref[b * SQ:(b + 1) * SQ, h * DH:(h + 1) * DH] = (
                    (csum[:, h * DH:(h + 1) * DH] * inv).astype(jnp.bfloat16)
                )

        wo_copy.wait()
        out_ref[...] = jnp.dot(
            ctx_ref[...], wo_ref[...].astype(jnp.bfloat16),
            preferred_element_type=jnp.float32,
        ).astype(jnp.bfloat16)
        out_copy = pltpu.make_async_copy(out_ref, out_hbm, out_sem)
        out_copy.start()
        out_copy.wait()

        for r in k_rdmas:
            r.wait_send()
        v_rdma.wait_send()

    n_msg = B * HQ + 1
    out = pl.pallas_call(
        body,
        out_shape=jax.ShapeDtypeStruct((B * SQ, 512), jnp.bfloat16),
        in_specs=[
            pl.BlockSpec(memory_space=pl.ANY),
            pl.BlockSpec(memory_space=pl.ANY),
            pl.BlockSpec(memory_space=pltpu.VMEM),
            pl.BlockSpec(memory_space=pl.ANY),
        ],
        out_specs=pl.BlockSpec(memory_space=pl.ANY),
        scratch_shapes=[
            pltpu.VMEM((B, HD, HS), jnp.bfloat16),
            pltpu.VMEM((B, HS, HD), jnp.bfloat16),
            pltpu.VMEM((B, HD, HS), jnp.bfloat16),
            pltpu.VMEM((B, HS, HD), jnp.bfloat16),
            pltpu.VMEM((B, SQ, HD), jnp.bfloat16),
            pltpu.VMEM((B, SQ, HD), jnp.bfloat16),
            pltpu.VMEM((B * SQ, HD), jnp.bfloat16),
            pltpu.VMEM((HD, 512), jnp.float32),
            pltpu.VMEM((B, SQ, 512), jnp.float32),
            pltpu.VMEM((512, HD), jnp.float32),
            pltpu.VMEM((B * SQ, 512), jnp.bfloat16),
            pltpu.SemaphoreType.DMA((n_msg,)),
            pltpu.SemaphoreType.DMA((n_msg,)),
            pltpu.SemaphoreType.DMA,
            pltpu.SemaphoreType.DMA,
            pltpu.SemaphoreType.DMA,
            pltpu.SemaphoreType.DMA,
        ],
        compiler_params=pltpu.CompilerParams(collective_id=0),
    )(x, Wq, kvT, Wo)
    return out.reshape(B, SQ, 512)


# device time: 11512 ns/iter; 1.0205x vs baseline; 1.0205x over previous
import jax
import jax.numpy as jnp
from jax import lax
from jax.experimental import pallas as pl
from jax.experimental.pallas import tpu as pltpu

N_DEV = 4
B, SQ, HQ, DH = 2, 128, 4, 64
QBLK = 64
NQB = SQ // QBLK


def kernel(x, Wq, K_ext, V_ext, Wo):
    kvT = jnp.concatenate(
        [K_ext.transpose(0, 2, 3, 1), V_ext.transpose(0, 2, 3, 1)], axis=0
    ).astype(jnp.bfloat16)

    def body(x_hbm, wq_hbm, kv_ref, wo_hbm, out_hbm,
             krem_ref, vrem_ref, ctx_ref, wo_ref, x_ref, wq_ref, out_ref,
             send_sems, recv_sems, wo_sem, x_sem, wq_sem, out_sem):
        k_ref = kv_ref.at[0:B]
        v_ref = kv_ref.at[B:2 * B]

        my = lax.axis_index("i")
        partner = (my + 2) % N_DEV

        x_copy = pltpu.make_async_copy(x_hbm, x_ref, x_sem)
        wq_copy = pltpu.make_async_copy(wq_hbm, wq_ref, wq_sem)
        wo_copy = pltpu.make_async_copy(wo_hbm, wo_ref, wo_sem)
        x_copy.start()
        wq_copy.start()
        wo_copy.start()

        barrier_sem = pltpu.get_barrier_semaphore()
        pl.semaphore_signal(
            barrier_sem, inc=1,
            device_id=(partner,), device_id_type=pl.DeviceIdType.MESH,
        )
        pl.semaphore_wait(barrier_sem, 1)

        k_rdma = pltpu.make_async_remote_copy(
            src_ref=k_ref, dst_ref=krem_ref,
            send_sem=send_sems.at[0], recv_sem=recv_sems.at[0],
            device_id=(partner,), device_id_type=pl.DeviceIdType.MESH,
        )
        v_rdma = pltpu.make_async_remote_copy(
            src_ref=v_ref, dst_ref=vrem_ref,
            send_sem=send_sems.at[1], recv_sem=recv_sems.at[1],
            device_id=(partner,), device_id_type=pl.DeviceIdType.MESH,
        )
        k_rdma.start()
        v_rdma.start()

        x_copy.wait()
        wq_copy.wait()
        q_all = jnp.dot(
            x_ref[...].reshape(B * SQ, 512).astype(jnp.bfloat16),
            wq_ref[...].astype(jnp.bfloat16),
            preferred_element_type=jnp.float32,
        )
        q_s = (q_all * 0.125).astype(jnp.bfloat16)

        def qblk(b, j, h):
            r0 = b * SQ + j * QBLK
            return q_s[r0:r0 + QBLK, h * DH:(h + 1) * DH]

        def tblk(ref, b, j, h):
            return ref[b, h, :, j * QBLK:(j + 1) * QBLK]

        loc = {}
        for b in range(B):
            for h in range(HQ):
                for j in range(NQB):
                    s_l = lax.dot_general(
                        qblk(b, j, h), tblk(k_ref, b, j, h),
                        (((1,), (0,)), ((), ())),
                        preferred_element_type=jnp.float32,
                    )
                    e_l = jnp.exp(s_l)
                    d_l = jnp.sum(e_l, axis=-1, keepdims=True)
                    c_l = lax.dot_general(
                        e_l.astype(jnp.bfloat16), tblk(v_ref, b, j, h),
                        (((1,), (1,)), ((), ())),
                        preferred_element_type=jnp.float32,
                    )
                    loc[b, h, j] = (d_l, c_l)

        k_rdma.wait_recv()
        rem = {}
        for b in range(B):
            for h in range(HQ):
                for j in range(NQB):
                    s_r = lax.dot_general(
                        qblk(b, j, h), tblk(krem_ref, b, j, h),
                        (((1,), (0,)), ((), ())),
                        preferred_element_type=jnp.float32,
                    )
                    e_r = jnp.exp(s_r)
                    d_r = jnp.sum(e_r, axis=-1, keepdims=True)
                    rem[b, h, j] = (e_r.astype(jnp.bfloat16), d_r)

        v_rdma.wait_recv()
        for b in range(B):
            for h in range(HQ):
                for j in range(NQB):
                    d_l, c_l = loc[b, h, j]
                    e_r, d_r = rem[b, h, j]
                    c_r = lax.dot_general(
                        e_r, tblk(vrem_ref, b, j, h),
                        (((1,), (1,)), ((), ())),
                        preferred_element_type=jnp.float32,
                    )
                    c = (c_l + c_r) * (1.0 / (d_l + d_r))
                    r0 = b * SQ + j * QBLK
                    ctx_ref[r0:r0 + QBLK, h * DH:(h + 1) * DH] = (
                        c.astype(jnp.bfloat16)
                    )

        wo_copy.wait()
        out_ref[...] = jnp.dot(
            ctx_ref[...], wo_ref[...].astype(jnp.bfloat16),
            preferred_element_type=jnp.float32,
        ).astype(jnp.bfloat16)
        out_copy = pltpu.make_async_copy(out_ref, out_hbm, out_sem)
        out_copy.start()
        out_copy.wait()

        k_rdma.wait_send()
        v_rdma.wait_send()

    out = pl.pallas_call(
        body,
        out_shape=jax.ShapeDtypeStruct((B * SQ, 512), jnp.bfloat16),
        in_specs=[
            pl.BlockSpec(memory_space=pl.ANY),
            pl.BlockSpec(memory_space=pl.ANY),
            pl.BlockSpec(memory_space=pltpu.VMEM),
            pl.BlockSpec(memory_space=pl.ANY),
        ],
        out_specs=pl.BlockSpec(memory_space=pl.ANY),
        scratch_shapes=[
            pltpu.VMEM((B, HQ, DH, SQ), jnp.bfloat16),
            pltpu.VMEM((B, HQ, DH, SQ), jnp.bfloat16),
            pltpu.VMEM((B * SQ, HQ * DH), jnp.bfloat16),
            pltpu.VMEM((HQ * DH, 512), jnp.float32),
            pltpu.VMEM((B, SQ, 512), jnp.float32),
            pltpu.VMEM((512, HQ * DH), jnp.float32),
            pltpu.VMEM((B * SQ, 512), jnp.bfloat16),
            pltpu.SemaphoreType.DMA((2,)),
            pltpu.SemaphoreType.DMA((2,)),
            pltpu.SemaphoreType.DMA,
            pltpu.SemaphoreType.DMA,
            pltpu.SemaphoreType.DMA,
            pltpu.SemaphoreType.DMA,
        ],
        compiler_params=pltpu.CompilerParams(collective_id=0),
    )(x, Wq, kvT, Wo)
    return out.reshape(B, SQ, 512)
